# Initial kernel scaffold; baseline (speedup 1.0000x reference)
import jax
import jax.numpy as jnp
from jax.experimental import pallas as pl


def kernel(x, edge_index, W1, b1, W2, b2):
    raise NotImplementedError("write your pallas kernel here")


# speedup vs baseline: 13.0815x; 13.0815x over previous
import functools

import jax
import jax.numpy as jnp
from jax import lax
from jax.experimental import pallas as pl
from jax.experimental.pallas import tpu as pltpu
from jax.experimental.pallas import tpu_sc as plsc

_K = 80
_BLK = 1024


def _sc_mesh():
    return plsc.VectorSubcoreMesh(core_axis_name="c", subcore_axis_name="s")


def _make_deg_kernel(e_pad, np_rows):
    mesh = _sc_mesh()
    nc, ns = mesh.num_cores, mesh.num_subcores
    nw = nc * ns
    ept = e_pad // nw
    nchunk = ept // _K
    rpt = np_rows // ns

    @functools.partial(
        pl.kernel,
        out_type=jax.ShapeDtypeStruct((nc, np_rows), jnp.float32),
        mesh=mesh,
        scratch_types=[
            pltpu.VMEM((_K,), jnp.int32),
            pltpu.VMEM((_K,), jnp.float32),
            pltpu.VMEM((rpt,), jnp.float32),
            pltpu.VMEM_SHARED((np_rows,), jnp.float32),
        ],
    )
    def deg_kernel(dst_hbm, out_hbm, dst_v, ones_v, zbuf_v, deg_sh):
        c = lax.axis_index("c")
        s = lax.axis_index("s")
        w = s * nc + c
        zeros16 = jnp.zeros((16,), jnp.float32)
        ones16 = jnp.ones((16,), jnp.float32)

        def _z(k, carry):
            zbuf_v[pl.ds(k * 16, 16)] = zeros16
            return carry

        lax.fori_loop(0, rpt // 16, _z, 0)
        for j in range(_K // 16):
            ones_v[pl.ds(j * 16, 16)] = ones16
        row0 = s * rpt
        pltpu.sync_copy(zbuf_v, deg_sh.at[pl.ds(row0, rpt)])
        plsc.subcore_barrier()

        base = w * ept

        def _chunk(g, carry):
            pltpu.sync_copy(dst_hbm.at[pl.ds(base + g * _K, _K)], dst_v)
            pltpu.sync_copy(ones_v, deg_sh.at[dst_v], add=True)
            return carry

        lax.fori_loop(0, nchunk, _chunk, 0)
        plsc.subcore_barrier()
        pltpu.sync_copy(deg_sh.at[pl.ds(row0, rpt)],
                        out_hbm.at[c, pl.ds(row0, rpt)])

    return deg_kernel


def _make_edge_kernel(e_pad, np_rows, d):
    mesh = _sc_mesh()
    nc, ns = mesh.num_cores, mesh.num_subcores
    nw = nc * ns
    ept = e_pad // nw
    nchunk = ept // _K
    rpt = np_rows // ns
    zr = 128
    nz = rpt // zr

    @functools.partial(
        pl.kernel,
        out_type=jax.ShapeDtypeStruct((nc, np_rows, d), jnp.float32),
        mesh=mesh,
        scratch_types=[
            pltpu.VMEM((_K,), jnp.int32),
            pltpu.VMEM((_K,), jnp.int32),
            pltpu.VMEM((_K, d), jnp.float32),
            pltpu.VMEM((zr, d), jnp.float32),
            pltpu.VMEM_SHARED((np_rows, d), jnp.float32),
            pltpu.SemaphoreType.DMA,
        ],
    )
    def edge_kernel(src_hbm, dst_hbm, y_hbm, out_hbm,
                    src_v, dst_v, rows_v, zbuf_v, acc_sh, gsem):
        c = lax.axis_index("c")
        s = lax.axis_index("s")
        w = s * nc + c
        zeros16 = jnp.zeros((16,), jnp.float32)
        lanes = d // 16

        def _z(k, carry):
            zbuf_v[k // lanes, pl.ds((k % lanes) * 16, 16)] = zeros16
            return carry

        lax.fori_loop(0, zr * lanes, _z, 0)
        row0 = s * rpt
        for i in range(nz):
            pltpu.sync_copy(zbuf_v, acc_sh.at[pl.ds(row0 + i * zr, zr)])
        plsc.subcore_barrier()

        base = w * ept

        def _chunk(g, carry):
            off = base + g * _K
            pltpu.sync_copy(src_hbm.at[pl.ds(off, _K)], src_v)
            pltpu.sync_copy(dst_hbm.at[pl.ds(off, _K)], dst_v)
            pltpu.async_copy(y_hbm.at[src_v], rows_v, gsem).wait()
            pltpu.sync_copy(rows_v, acc_sh.at[dst_v], add=True)
            return carry

        lax.fori_loop(0, nchunk, _chunk, 0)
        plsc.subcore_barrier()
        pltpu.sync_copy(acc_sh.at[pl.ds(row0, rpt)],
                        out_hbm.at[c, pl.ds(row0, rpt)])

    return edge_kernel


def _tc1_body(x_ref, wt_ref, d0_ref, d1_ref, y_ref):
    dinv = lax.rsqrt(d0_ref[...] + d1_ref[...] + 1.0)
    h = jnp.dot(x_ref[...], wt_ref[...], preferred_element_type=jnp.float32)
    y_ref[...] = dinv * h


def _tc2_body(s0_ref, s1_ref, y_ref, wt_ref, b_ref, d0_ref, d1_ref, o_ref):
    dinv = lax.rsqrt(d0_ref[...] + d1_ref[...] + 1.0)
    o1 = dinv * (s0_ref[...] + s1_ref[...] + y_ref[...]) + b_ref[...]
    o_ref[...] = dinv * jnp.dot(o1, wt_ref[...],
                                preferred_element_type=jnp.float32)


def _tc3_body(s0_ref, s1_ref, y_ref, b_ref, d0_ref, d1_ref, o_ref):
    dinv = lax.rsqrt(d0_ref[...] + d1_ref[...] + 1.0)
    o_ref[...] = dinv * (s0_ref[...] + s1_ref[...] + y_ref[...]) + b_ref[...]


def _row_spec(d):
    return pl.BlockSpec((_BLK, d), lambda i: (i, 0))


def _full_spec(shape):
    return pl.BlockSpec(shape, lambda i: tuple(0 for _ in shape))


def _tc1(x, wt, d0, d1):
    np_rows, d = x.shape
    return pl.pallas_call(
        _tc1_body,
        grid=(np_rows // _BLK,),
        in_specs=[_row_spec(d), _full_spec((d, d)),
                  _row_spec(1), _row_spec(1)],
        out_specs=_row_spec(d),
        out_shape=jax.ShapeDtypeStruct((np_rows, d), jnp.float32),
    )(x, wt, d0, d1)


def _tc2(s0, s1, y, wt, b, d0, d1):
    np_rows, d = y.shape
    return pl.pallas_call(
        _tc2_body,
        grid=(np_rows // _BLK,),
        in_specs=[_row_spec(d), _row_spec(d), _row_spec(d),
                  _full_spec((d, d)), _full_spec((1, d)),
                  _row_spec(1), _row_spec(1)],
        out_specs=_row_spec(d),
        out_shape=jax.ShapeDtypeStruct((np_rows, d), jnp.float32),
    )(s0, s1, y, wt, b, d0, d1)


def _tc3(s0, s1, y, b, d0, d1):
    np_rows, d = y.shape
    return pl.pallas_call(
        _tc3_body,
        grid=(np_rows // _BLK,),
        in_specs=[_row_spec(d), _row_spec(d), _row_spec(d),
                  _full_spec((1, d)), _row_spec(1), _row_spec(1)],
        out_specs=_row_spec(d),
        out_shape=jax.ShapeDtypeStruct((np_rows, d), jnp.float32),
    )(s0, s1, y, b, d0, d1)


def kernel(x, edge_index, W1, b1, W2, b2):
    n, d = x.shape
    e = edge_index.shape[1]
    np_rows = -(-n // _BLK) * _BLK
    nw = 32
    e_pad = -(-e // (nw * _K)) * (nw * _K)

    src = edge_index[0]
    dst = edge_index[1]
    if e_pad > e:
        src = jnp.concatenate(
            [src, jnp.zeros((e_pad - e,), jnp.int32)])
        dst = jnp.concatenate(
            [dst, jnp.full((e_pad - e,), np_rows - 1, jnp.int32)])
    x_p = jnp.pad(x, ((0, np_rows - n), (0, 0)))

    degp = _make_deg_kernel(e_pad, np_rows)(dst, )
    d0 = degp[0].reshape(np_rows, 1)
    d1 = degp[1].reshape(np_rows, 1)

    edge_k = _make_edge_kernel(e_pad, np_rows, d)

    y1 = _tc1(x_p, W1.T, d0, d1)
    s1 = edge_k(src, dst, y1)
    y2 = _tc2(s1[0], s1[1], y1, W2.T, b1.reshape(1, d), d0, d1)
    s2 = edge_k(src, dst, y2)
    o2 = _tc3(s2[0], s2[1], y2, b2.reshape(1, d), d0, d1)
    return o2[:n]


# speedup vs baseline: 22.0459x; 1.6853x over previous
import functools

import jax
import jax.numpy as jnp
from jax import lax
from jax.experimental import pallas as pl
from jax.experimental.pallas import tpu as pltpu
from jax.experimental.pallas import tpu_sc as plsc

_K = 64
_BLK = 1024


def _sc_mesh():
    return plsc.VectorSubcoreMesh(core_axis_name="c", subcore_axis_name="s")


def _make_deg_kernel(nchunk, np_rows):
    mesh = _sc_mesh()
    nc, ns = mesh.num_cores, mesh.num_subcores
    rpt = np_rows // ns
    grp = 8
    ngrp = nchunk // grp

    @functools.partial(
        pl.kernel,
        out_type=jax.ShapeDtypeStruct((nc, np_rows), jnp.float32),
        mesh=mesh,
        scratch_types=[
            pltpu.VMEM((nchunk, _K), jnp.int32),
            pltpu.VMEM((_K,), jnp.float32),
            pltpu.VMEM((rpt,), jnp.float32),
            pltpu.VMEM_SHARED((np_rows,), jnp.float32),
            pltpu.SemaphoreType.DMA,
        ],
    )
    def deg_kernel(dst_hbm, out_hbm, didx_v, ones_v, zbuf_v, deg_sh, dsem):
        c = lax.axis_index("c")
        s = lax.axis_index("s")
        w = s * nc + c
        zeros16 = jnp.zeros((16,), jnp.float32)
        ones16 = jnp.ones((16,), jnp.float32)

        def _z(k, carry):
            zbuf_v[pl.ds(k * 16, 16)] = zeros16
            return carry

        lax.fori_loop(0, rpt // 16, _z, 0)
        for j in range(_K // 16):
            ones_v[pl.ds(j * 16, 16)] = ones16
        row0 = s * rpt
        pltpu.sync_copy(zbuf_v, deg_sh.at[pl.ds(row0, rpt)])
        pltpu.sync_copy(dst_hbm.at[w], didx_v)
        plsc.subcore_barrier()

        def _grp(p, carry):
            for j in range(grp):
                pltpu.async_copy(ones_v, deg_sh.at[didx_v.at[p * grp + j]],
                                 dsem, add=True)
            for j in range(grp):
                pltpu.make_async_copy(ones_v, deg_sh.at[didx_v.at[0]],
                                      dsem).wait()
            return carry

        lax.fori_loop(0, ngrp, _grp, 0)
        plsc.subcore_barrier()
        pltpu.sync_copy(deg_sh.at[pl.ds(row0, rpt)],
                        out_hbm.at[c, pl.ds(row0, rpt)])

    return deg_kernel


def _make_edge_kernel(nchunk, np_rows, d):
    mesh = _sc_mesh()
    nc, ns = mesh.num_cores, mesh.num_subcores
    rpt = np_rows // ns
    nz = rpt // _K
    nquad = nchunk // 4

    @functools.partial(
        pl.kernel,
        out_type=jax.ShapeDtypeStruct((nc, np_rows, d), jnp.float32),
        mesh=mesh,
        scratch_types=[
            pltpu.VMEM((4, _K), jnp.int32),
            pltpu.VMEM((4, _K), jnp.int32),
            pltpu.VMEM((_K, d), jnp.float32),
            pltpu.VMEM((_K, d), jnp.float32),
            pltpu.VMEM_SHARED((np_rows, d), jnp.float32),
            pltpu.SemaphoreType.DMA,
            pltpu.SemaphoreType.DMA,
            pltpu.SemaphoreType.DMA,
            pltpu.SemaphoreType.DMA,
            pltpu.SemaphoreType.DMA,
            pltpu.SemaphoreType.DMA,
        ],
    )
    def edge_kernel(src_hbm, dst_hbm, y_hbm, out_hbm,
                    sidx_v, didx_v, rows_a, rows_b, acc_sh,
                    isem_a, isem_b, gsem_a, gsem_b, ssem_a, ssem_b):
        c = lax.axis_index("c")
        s = lax.axis_index("s")
        w = s * nc + c
        zeros16 = jnp.zeros((16,), jnp.float32)
        lanes = d // 16

        def _z(k, carry):
            rows_a[k // lanes, pl.ds((k % lanes) * 16, 16)] = zeros16
            return carry

        lax.fori_loop(0, _K * lanes, _z, 0)
        row0 = s * rpt
        for i in range(nz):
            pltpu.sync_copy(rows_a, acc_sh.at[pl.ds(row0 + i * _K, _K)])
        pltpu.sync_copy(src_hbm.at[w, 0], sidx_v.at[0])
        pltpu.sync_copy(dst_hbm.at[w, 0], didx_v.at[0])
        pltpu.sync_copy(src_hbm.at[w, 1], sidx_v.at[1])
        pltpu.sync_copy(dst_hbm.at[w, 1], didx_v.at[1])
        plsc.subcore_barrier()

        def _sub(g, j, first):
            rows = rows_a if j % 2 == 0 else rows_b
            isem = isem_a if j % 2 == 0 else isem_b
            gsem = gsem_a if j % 2 == 0 else gsem_b
            ssem = ssem_a if j % 2 == 0 else ssem_b
            if not (first and j < 2):
                pltpu.make_async_copy(rows, acc_sh.at[didx_v.at[0]],
                                      ssem).wait()
                pltpu.make_async_copy(src_hbm.at[w, 0], sidx_v.at[0],
                                      isem).wait()
                pltpu.make_async_copy(dst_hbm.at[w, 0], didx_v.at[0],
                                      isem).wait()
            gn = jnp.minimum(g + 2, nchunk - 1)
            sn = (j + 2) % 4
            pltpu.async_copy(src_hbm.at[w, gn], sidx_v.at[sn], isem)
            pltpu.async_copy(dst_hbm.at[w, gn], didx_v.at[sn], isem)
            pltpu.async_copy(y_hbm.at[sidx_v.at[j]], rows, gsem).wait()
            pltpu.async_copy(rows, acc_sh.at[didx_v.at[j]], ssem, add=True)

        for j in range(4):
            _sub(j, j, True)

        def _quad(q, carry):
            for j in range(4):
                _sub(4 * q + j, j, False)
            return carry

        lax.fori_loop(1, nquad, _quad, 0)
        pltpu.make_async_copy(rows_a, acc_sh.at[didx_v.at[0]], ssem_a).wait()
        pltpu.make_async_copy(rows_b, acc_sh.at[didx_v.at[0]], ssem_b).wait()
        for isem in (isem_a, isem_b):
            pltpu.make_async_copy(src_hbm.at[w, 0], sidx_v.at[0],
                                  isem).wait()
            pltpu.make_async_copy(src_hbm.at[w, 0], sidx_v.at[0],
                                  isem).wait()
        plsc.subcore_barrier()
        pltpu.sync_copy(acc_sh.at[pl.ds(row0, rpt)],
                        out_hbm.at[c, pl.ds(row0, rpt)])

    return edge_kernel


def _tc1_body(x_ref, wt_ref, d0_ref, d1_ref, y_ref):
    dinv = lax.rsqrt(d0_ref[...] + d1_ref[...] + 1.0)
    h = jnp.dot(x_ref[...], wt_ref[...], preferred_element_type=jnp.float32)
    y_ref[...] = dinv * h


def _tc2_body(s0_ref, s1_ref, y_ref, wt_ref, b_ref, d0_ref, d1_ref, o_ref):
    dinv = lax.rsqrt(d0_ref[...] + d1_ref[...] + 1.0)
    o1 = dinv * (s0_ref[...] + s1_ref[...] + y_ref[...]) + b_ref[...]
    o_ref[...] = dinv * jnp.dot(o1, wt_ref[...],
                                preferred_element_type=jnp.float32)


def _tc3_body(s0_ref, s1_ref, y_ref, b_ref, d0_ref, d1_ref, o_ref):
    dinv = lax.rsqrt(d0_ref[...] + d1_ref[...] + 1.0)
    o_ref[...] = dinv * (s0_ref[...] + s1_ref[...] + y_ref[...]) + b_ref[...]


def _row_spec(d):
    return pl.BlockSpec((_BLK, d), lambda i: (i, 0))


def _full_spec(shape):
    return pl.BlockSpec(shape, lambda i: tuple(0 for _ in shape))


def _tc1(x, wt, d0, d1):
    np_rows, d = x.shape
    return pl.pallas_call(
        _tc1_body,
        grid=(np_rows // _BLK,),
        in_specs=[_row_spec(d), _full_spec((d, d)),
                  _row_spec(1), _row_spec(1)],
        out_specs=_row_spec(d),
        out_shape=jax.ShapeDtypeStruct((np_rows, d), jnp.float32),
    )(x, wt, d0, d1)


def _tc2(s0, s1, y, wt, b, d0, d1):
    np_rows, d = y.shape
    return pl.pallas_call(
        _tc2_body,
        grid=(np_rows // _BLK,),
        in_specs=[_row_spec(d), _row_spec(d), _row_spec(d),
                  _full_spec((d, d)), _full_spec((1, d)),
                  _row_spec(1), _row_spec(1)],
        out_specs=_row_spec(d),
        out_shape=jax.ShapeDtypeStruct((np_rows, d), jnp.float32),
    )(s0, s1, y, wt, b, d0, d1)


def _tc3(s0, s1, y, b, d0, d1):
    np_rows, d = y.shape
    return pl.pallas_call(
        _tc3_body,
        grid=(np_rows // _BLK,),
        in_specs=[_row_spec(d), _row_spec(d), _row_spec(d),
                  _full_spec((1, d)), _row_spec(1), _row_spec(1)],
        out_specs=_row_spec(d),
        out_shape=jax.ShapeDtypeStruct((np_rows, d), jnp.float32),
    )(s0, s1, y, b, d0, d1)


def kernel(x, edge_index, W1, b1, W2, b2):
    n, d = x.shape
    e = edge_index.shape[1]
    np_rows = -(-n // _BLK) * _BLK
    if np_rows == n:
        np_rows += _BLK
    nw = 32
    ept = -(-e // nw // (4 * _K)) * (4 * _K)
    e_pad = ept * nw
    nchunk = ept // _K

    src = edge_index[0]
    dst = edge_index[1]
    if e_pad > e:
        pad = e_pad - e
        src = jnp.concatenate([src, (jnp.arange(pad, dtype=jnp.int32) % n)])
        dst = jnp.concatenate(
            [dst, n + (jnp.arange(pad, dtype=jnp.int32) % (np_rows - n))])
    src3 = src.reshape(nw, nchunk, _K)
    dst3 = dst.reshape(nw, nchunk, _K)
    x_p = jnp.pad(x, ((0, np_rows - n), (0, 0)))

    degp = _make_deg_kernel(nchunk, np_rows)(dst3)
    d0 = degp[0].reshape(np_rows, 1)
    d1 = degp[1].reshape(np_rows, 1)

    edge_k = _make_edge_kernel(nchunk, np_rows, d)

    y1 = _tc1(x_p, W1.T, d0, d1)
    s1 = edge_k(src3, dst3, y1)
    y2 = _tc2(s1[0], s1[1], y1, W2.T, b1.reshape(1, d), d0, d1)
    s2 = edge_k(src3, dst3, y2)
    o2 = _tc3(s2[0], s2[1], y2, b2.reshape(1, d), d0, d1)
    return o2[:n]


# speedup vs baseline: 31.4187x; 1.4251x over previous
import functools

import jax
import jax.numpy as jnp
from jax import lax
from jax.experimental import pallas as pl
from jax.experimental.pallas import tpu as pltpu
from jax.experimental.pallas import tpu_sc as plsc

_K = 64
_BLK = 1024


def _sc_mesh():
    return plsc.VectorSubcoreMesh(core_axis_name="c", subcore_axis_name="s")


def _make_deg_kernel(nchunk, np_rows):
    mesh = _sc_mesh()
    nc, ns = mesh.num_cores, mesh.num_subcores
    rpt = np_rows // ns
    grp = 8
    ngrp = nchunk // grp

    @functools.partial(
        pl.kernel,
        out_type=jax.ShapeDtypeStruct((nc, np_rows), jnp.float32),
        mesh=mesh,
        scratch_types=[
            pltpu.VMEM((nchunk, _K), jnp.int32),
            pltpu.VMEM((_K,), jnp.float32),
            pltpu.VMEM((rpt,), jnp.float32),
            pltpu.VMEM_SHARED((np_rows,), jnp.float32),
            pltpu.SemaphoreType.DMA,
        ],
    )
    def deg_kernel(dst_hbm, out_hbm, didx_v, ones_v, zbuf_v, deg_sh, dsem):
        c = lax.axis_index("c")
        s = lax.axis_index("s")
        w = s * nc + c
        zeros16 = jnp.zeros((16,), jnp.float32)
        ones16 = jnp.ones((16,), jnp.float32)

        def _z(k, carry):
            zbuf_v[pl.ds(k * 16, 16)] = zeros16
            return carry

        lax.fori_loop(0, rpt // 16, _z, 0)
        for j in range(_K // 16):
            ones_v[pl.ds(j * 16, 16)] = ones16
        row0 = s * rpt
        pltpu.sync_copy(zbuf_v, deg_sh.at[pl.ds(row0, rpt)])
        pltpu.sync_copy(dst_hbm.at[w], didx_v)
        plsc.subcore_barrier()

        def _grp(p, carry):
            for j in range(grp):
                pltpu.async_copy(ones_v, deg_sh.at[didx_v.at[p * grp + j]],
                                 dsem, add=True)
            for j in range(grp):
                pltpu.make_async_copy(ones_v, deg_sh.at[didx_v.at[0]],
                                      dsem).wait()
            return carry

        lax.fori_loop(0, ngrp, _grp, 0)
        plsc.subcore_barrier()
        pltpu.sync_copy(deg_sh.at[pl.ds(row0, rpt)],
                        out_hbm.at[c, pl.ds(row0, rpt)])

    return deg_kernel


def _make_edge_kernel(nchunk, np_rows, d):
    mesh = _sc_mesh()
    nc, ns = mesh.num_cores, mesh.num_subcores
    rpt = np_rows // ns
    nz = rpt // _K
    noct = nchunk // 8

    @functools.partial(
        pl.kernel,
        out_type=jax.ShapeDtypeStruct((nc, np_rows, d), jnp.float32),
        mesh=mesh,
        scratch_types=[
            pltpu.VMEM((8, 2, _K), jnp.int32),
            [pltpu.VMEM((_K, d), jnp.float32) for _ in range(4)],
            pltpu.VMEM_SHARED((np_rows, d), jnp.float32),
            [pltpu.SemaphoreType.DMA for _ in range(4)],
            [pltpu.SemaphoreType.DMA for _ in range(4)],
            [pltpu.SemaphoreType.DMA for _ in range(4)],
        ],
    )
    def edge_kernel(idx_hbm, y_hbm, out_hbm,
                    ring, rows, acc_sh, isem, gsem, ssem):
        c = lax.axis_index("c")
        s = lax.axis_index("s")
        w = s * nc + c
        zeros16 = jnp.zeros((16,), jnp.float32)
        lanes = d // 16

        def _z(k, carry):
            rows[0][k // lanes, pl.ds((k % lanes) * 16, 16)] = zeros16
            return carry

        lax.fori_loop(0, _K * lanes, _z, 0)
        row0 = s * rpt
        for i in range(nz):
            pltpu.sync_copy(rows[0], acc_sh.at[pl.ds(row0 + i * _K, _K)])
        for j in range(4):
            pltpu.sync_copy(idx_hbm.at[w, j], ring.at[j])
        plsc.subcore_barrier()

        def _sub(g, j, first):
            m = j % 4
            if not (first and j < 4):
                pltpu.make_async_copy(rows[m], acc_sh.at[ring.at[0, 1]],
                                      ssem[m]).wait()
                pltpu.make_async_copy(idx_hbm.at[w, 0], ring.at[0],
                                      isem[m]).wait()
            gn = jnp.minimum(g + 4, nchunk - 1)
            pltpu.async_copy(idx_hbm.at[w, gn], ring.at[(j + 4) % 8],
                             isem[m])
            pltpu.async_copy(y_hbm.at[ring.at[j, 0]], rows[m], gsem[m])
            if not (first and j < 2):
                m2 = (j + 2) % 4
                pltpu.make_async_copy(y_hbm.at[ring.at[0, 0]], rows[m2],
                                      gsem[m2]).wait()
                pltpu.async_copy(rows[m2], acc_sh.at[ring.at[(j + 6) % 8, 1]],
                                 ssem[m2], add=True)

        for j in range(8):
            _sub(j, j, True)

        def _oct(q, carry):
            for j in range(8):
                _sub(8 * q + j, j, False)
            return carry

        lax.fori_loop(1, noct, _oct, 0)
        for (m2, slot) in ((2, 6), (3, 7)):
            pltpu.make_async_copy(y_hbm.at[ring.at[0, 0]], rows[m2],
                                  gsem[m2]).wait()
            pltpu.async_copy(rows[m2], acc_sh.at[ring.at[slot, 1]],
                             ssem[m2], add=True)
        for m in range(4):
            pltpu.make_async_copy(rows[m], acc_sh.at[ring.at[0, 1]],
                                  ssem[m]).wait()
            pltpu.make_async_copy(idx_hbm.at[w, 0], ring.at[0],
                                  isem[m]).wait()
        plsc.subcore_barrier()
        pltpu.sync_copy(acc_sh.at[pl.ds(row0, rpt)],
                        out_hbm.at[c, pl.ds(row0, rpt)])

    return edge_kernel


def _tc1_body(x_ref, wt_ref, d0_ref, d1_ref, y_ref):
    dinv = lax.rsqrt(d0_ref[...] + d1_ref[...] + 1.0)
    h = jnp.dot(x_ref[...], wt_ref[...], preferred_element_type=jnp.float32)
    y_ref[...] = dinv * h


def _tc2_body(s0_ref, s1_ref, y_ref, wt_ref, b_ref, d0_ref, d1_ref, o_ref):
    dinv = lax.rsqrt(d0_ref[...] + d1_ref[...] + 1.0)
    o1 = dinv * (s0_ref[...] + s1_ref[...] + y_ref[...]) + b_ref[...]
    o_ref[...] = dinv * jnp.dot(o1, wt_ref[...],
                                preferred_element_type=jnp.float32)


def _tc3_body(s0_ref, s1_ref, y_ref, b_ref, d0_ref, d1_ref, o_ref):
    dinv = lax.rsqrt(d0_ref[...] + d1_ref[...] + 1.0)
    o_ref[...] = dinv * (s0_ref[...] + s1_ref[...] + y_ref[...]) + b_ref[...]


def _row_spec(d):
    return pl.BlockSpec((_BLK, d), lambda i: (i, 0))


def _full_spec(shape):
    return pl.BlockSpec(shape, lambda i: tuple(0 for _ in shape))


def _tc1(x, wt, d0, d1):
    np_rows, d = x.shape
    return pl.pallas_call(
        _tc1_body,
        grid=(np_rows // _BLK,),
        in_specs=[_row_spec(d), _full_spec((d, d)),
                  _row_spec(1), _row_spec(1)],
        out_specs=_row_spec(d),
        out_shape=jax.ShapeDtypeStruct((np_rows, d), jnp.float32),
    )(x, wt, d0, d1)


def _tc2(s0, s1, y, wt, b, d0, d1):
    np_rows, d = y.shape
    return pl.pallas_call(
        _tc2_body,
        grid=(np_rows // _BLK,),
        in_specs=[_row_spec(d), _row_spec(d), _row_spec(d),
                  _full_spec((d, d)), _full_spec((1, d)),
                  _row_spec(1), _row_spec(1)],
        out_specs=_row_spec(d),
        out_shape=jax.ShapeDtypeStruct((np_rows, d), jnp.float32),
    )(s0, s1, y, wt, b, d0, d1)


def _tc3(s0, s1, y, b, d0, d1):
    np_rows, d = y.shape
    return pl.pallas_call(
        _tc3_body,
        grid=(np_rows // _BLK,),
        in_specs=[_row_spec(d), _row_spec(d), _row_spec(d),
                  _full_spec((1, d)), _row_spec(1), _row_spec(1)],
        out_specs=_row_spec(d),
        out_shape=jax.ShapeDtypeStruct((np_rows, d), jnp.float32),
    )(s0, s1, y, b, d0, d1)


def kernel(x, edge_index, W1, b1, W2, b2):
    n, d = x.shape
    e = edge_index.shape[1]
    np_rows = -(-n // _BLK) * _BLK
    if np_rows == n:
        np_rows += _BLK
    nw = 32
    ept = -(-e // nw // (8 * _K)) * (8 * _K)
    e_pad = ept * nw
    nchunk = ept // _K

    src = edge_index[0]
    dst = edge_index[1]
    if e_pad > e:
        pad = e_pad - e
        src = jnp.concatenate([src, (jnp.arange(pad, dtype=jnp.int32) % n)])
        dst = jnp.concatenate(
            [dst, n + (jnp.arange(pad, dtype=jnp.int32) % (np_rows - n))])
    src3 = src.reshape(nw, nchunk, _K)
    dst3 = dst.reshape(nw, nchunk, _K)
    idx4 = jnp.stack([src3, dst3], axis=2)
    x_p = jnp.pad(x, ((0, np_rows - n), (0, 0)))

    degp = _make_deg_kernel(nchunk, np_rows)(dst3)
    d0 = degp[0].reshape(np_rows, 1)
    d1 = degp[1].reshape(np_rows, 1)

    edge_k = _make_edge_kernel(nchunk, np_rows, d)

    y1 = _tc1(x_p, W1.T, d0, d1)
    s1 = edge_k(idx4, y1)
    y2 = _tc2(s1[0], s1[1], y1, W2.T, b1.reshape(1, d), d0, d1)
    s2 = edge_k(idx4, y2)
    o2 = _tc3(s2[0], s2[1], y2, b2.reshape(1, d), d0, d1)
    return o2[:n]


# speedup vs baseline: 32.2598x; 1.0268x over previous
import functools

import jax
import jax.numpy as jnp
from jax import lax
from jax.experimental import pallas as pl
from jax.experimental.pallas import tpu as pltpu
from jax.experimental.pallas import tpu_sc as plsc

_K = 64
_BLK = 1024


def _sc_mesh():
    return plsc.VectorSubcoreMesh(core_axis_name="c", subcore_axis_name="s")


def _make_deg_kernel(nchunk, np_rows):
    mesh = _sc_mesh()
    nc, ns = mesh.num_cores, mesh.num_subcores
    rpt = np_rows // ns
    grp = 8
    ngrp = nchunk // grp

    @functools.partial(
        pl.kernel,
        out_type=jax.ShapeDtypeStruct((nc, np_rows), jnp.float32),
        mesh=mesh,
        scratch_types=[
            pltpu.VMEM((nchunk, _K), jnp.int32),
            pltpu.VMEM((_K,), jnp.float32),
            pltpu.VMEM((rpt,), jnp.float32),
            pltpu.VMEM_SHARED((np_rows,), jnp.float32),
            pltpu.SemaphoreType.DMA,
        ],
    )
    def deg_kernel(dst_hbm, out_hbm, didx_v, ones_v, zbuf_v, deg_sh, dsem):
        c = lax.axis_index("c")
        s = lax.axis_index("s")
        w = s * nc + c
        zeros16 = jnp.zeros((16,), jnp.float32)
        ones16 = jnp.ones((16,), jnp.float32)

        def _z(k, carry):
            zbuf_v[pl.ds(k * 16, 16)] = zeros16
            return carry

        lax.fori_loop(0, rpt // 16, _z, 0)
        for j in range(_K // 16):
            ones_v[pl.ds(j * 16, 16)] = ones16
        row0 = s * rpt
        pltpu.sync_copy(zbuf_v, deg_sh.at[pl.ds(row0, rpt)])
        pltpu.sync_copy(dst_hbm.at[w], didx_v)
        plsc.subcore_barrier()

        def _grp(p, carry):
            for j in range(grp):
                pltpu.async_copy(ones_v, deg_sh.at[didx_v.at[p * grp + j]],
                                 dsem, add=True)
            for j in range(grp):
                pltpu.make_async_copy(ones_v, deg_sh.at[didx_v.at[0]],
                                      dsem).wait()
            return carry

        lax.fori_loop(0, ngrp, _grp, 0)
        plsc.subcore_barrier()
        pltpu.sync_copy(deg_sh.at[pl.ds(row0, rpt)],
                        out_hbm.at[c, pl.ds(row0, rpt)])

    return deg_kernel


def _make_edge_kernel(nchunk, np_rows, d):
    mesh = _sc_mesh()
    nc, ns = mesh.num_cores, mesh.num_subcores
    rpt = np_rows // ns
    nz = rpt // _K
    noct = nchunk // 8

    @functools.partial(
        pl.kernel,
        out_type=jax.ShapeDtypeStruct((nc, np_rows, d), jnp.float32),
        mesh=mesh,
        scratch_types=[
            pltpu.VMEM((8, 2, _K), jnp.int32),
            [pltpu.VMEM((_K, d), jnp.float32) for _ in range(4)],
            pltpu.VMEM_SHARED((np_rows, d), jnp.float32),
            [pltpu.SemaphoreType.DMA for _ in range(4)],
            [pltpu.SemaphoreType.DMA for _ in range(4)],
            [pltpu.SemaphoreType.DMA for _ in range(4)],
        ],
    )
    def edge_kernel(idx_hbm, y_hbm, out_hbm,
                    ring, rows, acc_sh, isem, gsem, ssem):
        c = lax.axis_index("c")
        s = lax.axis_index("s")
        w = s * nc + c
        zeros16 = jnp.zeros((16,), jnp.float32)
        lanes = d // 16

        def _z(k, carry):
            rows[0][k // lanes, pl.ds((k % lanes) * 16, 16)] = zeros16
            return carry

        lax.fori_loop(0, _K * lanes, _z, 0)
        row0 = s * rpt
        for i in range(nz):
            pltpu.sync_copy(rows[0], acc_sh.at[pl.ds(row0 + i * _K, _K)])
        for j in range(4):
            pltpu.sync_copy(idx_hbm.at[w, j], ring.at[j])
        plsc.subcore_barrier()

        def _sub(g, j, first):
            m = j % 4
            if not (first and j < 4):
                pltpu.make_async_copy(rows[m], acc_sh.at[ring.at[0, 1]],
                                      ssem[m]).wait()
                pltpu.make_async_copy(idx_hbm.at[w, 0], ring.at[0],
                                      isem[m]).wait()
            gn = jnp.minimum(g + 4, nchunk - 1)
            pltpu.async_copy(idx_hbm.at[w, gn], ring.at[(j + 4) % 8],
                             isem[m])
            pltpu.async_copy(y_hbm.at[ring.at[j, 0]], rows[m], gsem[m])
            if not (first and j < 2):
                m2 = (j + 2) % 4
                pltpu.make_async_copy(y_hbm.at[ring.at[0, 0]], rows[m2],
                                      gsem[m2]).wait()
                pltpu.async_copy(rows[m2], acc_sh.at[ring.at[(j + 6) % 8, 1]],
                                 ssem[m2], add=True)

        for j in range(8):
            _sub(j, j, True)

        def _oct(q, carry):
            for j in range(8):
                _sub(8 * q + j, j, False)
            return carry

        lax.fori_loop(1, noct, _oct, 0)
        for (m2, slot) in ((2, 6), (3, 7)):
            pltpu.make_async_copy(y_hbm.at[ring.at[0, 0]], rows[m2],
                                  gsem[m2]).wait()
            pltpu.async_copy(rows[m2], acc_sh.at[ring.at[slot, 1]],
                             ssem[m2], add=True)
        for m in range(4):
            pltpu.make_async_copy(rows[m], acc_sh.at[ring.at[0, 1]],
                                  ssem[m]).wait()
            pltpu.make_async_copy(idx_hbm.at[w, 0], ring.at[0],
                                  isem[m]).wait()
        plsc.subcore_barrier()
        pltpu.sync_copy(acc_sh.at[pl.ds(row0, rpt)],
                        out_hbm.at[c, pl.ds(row0, rpt)])

    return edge_kernel


def _matmul_t(a, w):
    return lax.dot_general(a, w, (((1,), (1,)), ((), ())),
                           preferred_element_type=jnp.float32)


def _tc1_body(x_ref, wt_ref, d0_ref, d1_ref, y_ref):
    dinv = lax.rsqrt(d0_ref[...] + d1_ref[...] + 1.0)
    y_ref[...] = dinv * _matmul_t(x_ref[...], wt_ref[...])


def _tc2_body(s0_ref, s1_ref, y_ref, wt_ref, b_ref, d0_ref, d1_ref, o_ref):
    dinv = lax.rsqrt(d0_ref[...] + d1_ref[...] + 1.0)
    o1 = dinv * (s0_ref[...] + s1_ref[...] + y_ref[...]) + b_ref[...]
    o_ref[...] = dinv * _matmul_t(o1, wt_ref[...])


def _tc3_body(s0_ref, s1_ref, y_ref, b_ref, d0_ref, d1_ref, o_ref):
    dinv = lax.rsqrt(d0_ref[...] + d1_ref[...] + 1.0)
    o_ref[...] = dinv * (s0_ref[...] + s1_ref[...] + y_ref[...]) + b_ref[...]


def _row_spec(d):
    return pl.BlockSpec((_BLK, d), lambda i: (i, 0))


def _full_spec(shape):
    return pl.BlockSpec(shape, lambda i: tuple(0 for _ in shape))


def _tc1(x, wt, d0, d1, np_rows):
    _, d = x.shape
    return pl.pallas_call(
        _tc1_body,
        grid=(np_rows // _BLK,),
        in_specs=[_row_spec(d), _full_spec((d, d)),
                  _row_spec(1), _row_spec(1)],
        out_specs=_row_spec(d),
        out_shape=jax.ShapeDtypeStruct((np_rows, d), jnp.float32),
    )(x, wt, d0, d1)


def _tc2(s0, s1, y, wt, b, d0, d1):
    np_rows, d = y.shape
    return pl.pallas_call(
        _tc2_body,
        grid=(np_rows // _BLK,),
        in_specs=[_row_spec(d), _row_spec(d), _row_spec(d),
                  _full_spec((d, d)), _full_spec((1, d)),
                  _row_spec(1), _row_spec(1)],
        out_specs=_row_spec(d),
        out_shape=jax.ShapeDtypeStruct((np_rows, d), jnp.float32),
    )(s0, s1, y, wt, b, d0, d1)


def _tc3(s0, s1, y, b, d0, d1, n):
    np_rows, d = y.shape
    return pl.pallas_call(
        _tc3_body,
        grid=(np_rows // _BLK,),
        in_specs=[_row_spec(d), _row_spec(d), _row_spec(d),
                  _full_spec((1, d)), _row_spec(1), _row_spec(1)],
        out_specs=_row_spec(d),
        out_shape=jax.ShapeDtypeStruct((n, d), jnp.float32),
    )(s0, s1, y, b, d0, d1)


def kernel(x, edge_index, W1, b1, W2, b2):
    n, d = x.shape
    e = edge_index.shape[1]
    np_rows = -(-n // _BLK) * _BLK
    if np_rows == n:
        np_rows += _BLK
    nw = 32
    ept = -(-e // nw // (8 * _K)) * (8 * _K)
    e_pad = ept * nw
    nchunk = ept // _K

    src = edge_index[0]
    dst = edge_index[1]
    if e_pad > e:
        pad = e_pad - e
        src = jnp.concatenate([src, (jnp.arange(pad, dtype=jnp.int32) % n)])
        dst = jnp.concatenate(
            [dst, n + (jnp.arange(pad, dtype=jnp.int32) % (np_rows - n))])
    src3 = src.reshape(nw, nchunk, _K)
    dst3 = dst.reshape(nw, nchunk, _K)
    idx4 = jnp.stack([src3, dst3], axis=2)

    degp = _make_deg_kernel(nchunk, np_rows)(dst3)
    d0 = degp[0].reshape(np_rows, 1)
    d1 = degp[1].reshape(np_rows, 1)

    edge_k = _make_edge_kernel(nchunk, np_rows, d)

    y1 = _tc1(x, W1, d0, d1, np_rows)
    s1 = edge_k(idx4, y1)
    y2 = _tc2(s1[0], s1[1], y1, W2, b1.reshape(1, d), d0, d1)
    s2 = edge_k(idx4, y2)
    return _tc3(s2[0], s2[1], y2, b2.reshape(1, d), d0, d1, n)


# speedup vs baseline: 32.3884x; 1.0040x over previous
import functools

import jax
import jax.numpy as jnp
from jax import lax
from jax.experimental import pallas as pl
from jax.experimental.pallas import tpu as pltpu
from jax.experimental.pallas import tpu_sc as plsc

_K = 64
_BLK = 1024


def _sc_mesh():
    return plsc.VectorSubcoreMesh(core_axis_name="c", subcore_axis_name="s")


def _make_deg_kernel(nchunk, np_rows):
    mesh = _sc_mesh()
    nc, ns = mesh.num_cores, mesh.num_subcores
    rpt = np_rows // ns
    grp = 8
    ngrp = nchunk // grp

    @functools.partial(
        pl.kernel,
        out_type=jax.ShapeDtypeStruct((nc, np_rows), jnp.float32),
        mesh=mesh,
        scratch_types=[
            pltpu.VMEM((nchunk, _K), jnp.int32),
            pltpu.VMEM((_K,), jnp.float32),
            pltpu.VMEM((rpt,), jnp.float32),
            pltpu.VMEM_SHARED((np_rows,), jnp.float32),
            pltpu.SemaphoreType.DMA,
        ],
    )
    def deg_kernel(dst_hbm, out_hbm, didx_v, ones_v, zbuf_v, deg_sh, dsem):
        c = lax.axis_index("c")
        s = lax.axis_index("s")
        w = s * nc + c
        zeros16 = jnp.zeros((16,), jnp.float32)
        ones16 = jnp.ones((16,), jnp.float32)

        def _z(k, carry):
            zbuf_v[pl.ds(k * 16, 16)] = zeros16
            return carry

        lax.fori_loop(0, rpt // 16, _z, 0)
        for j in range(_K // 16):
            ones_v[pl.ds(j * 16, 16)] = ones16
        row0 = s * rpt
        pltpu.sync_copy(zbuf_v, deg_sh.at[pl.ds(row0, rpt)])
        pltpu.sync_copy(dst_hbm.at[w], didx_v)
        plsc.subcore_barrier()

        def _grp(p, carry):
            for j in range(grp):
                pltpu.async_copy(ones_v, deg_sh.at[didx_v.at[p * grp + j]],
                                 dsem, add=True)
            for j in range(grp):
                pltpu.make_async_copy(ones_v, deg_sh.at[didx_v.at[0]],
                                      dsem).wait()
            return carry

        lax.fori_loop(0, ngrp, _grp, 0)
        plsc.subcore_barrier()
        pltpu.sync_copy(deg_sh.at[pl.ds(row0, rpt)],
                        out_hbm.at[c, pl.ds(row0, rpt)])

    return deg_kernel


def _make_edge_kernel(nchunk, np_rows, d):
    mesh = _sc_mesh()
    nc, ns = mesh.num_cores, mesh.num_subcores
    rpt = np_rows // ns
    nz = rpt // _K
    noct = nchunk // 8

    @functools.partial(
        pl.kernel,
        out_type=jax.ShapeDtypeStruct((nc, np_rows, d), jnp.float32),
        mesh=mesh,
        scratch_types=[
            pltpu.VMEM((8, 2, _K), jnp.int32),
            [pltpu.VMEM((_K, d), jnp.float32) for _ in range(4)],
            pltpu.VMEM_SHARED((np_rows, d), jnp.float32),
            [pltpu.SemaphoreType.DMA for _ in range(4)],
            [pltpu.SemaphoreType.DMA for _ in range(4)],
            [pltpu.SemaphoreType.DMA for _ in range(4)],
        ],
    )
    def edge_kernel(idx_hbm, y_hbm, out_hbm,
                    ring, rows, acc_sh, isem, gsem, ssem):
        c = lax.axis_index("c")
        s = lax.axis_index("s")
        w = s * nc + c
        zeros16 = jnp.zeros((16,), jnp.float32)
        lanes = d // 16

        for j in range(4):
            pltpu.sync_copy(idx_hbm.at[w, j], ring.at[j])
        for j in range(4):
            pltpu.async_copy(idx_hbm.at[w, 4 + j], ring.at[4 + j],
                             isem[j])
        for m in (1, 2, 3):
            pltpu.async_copy(y_hbm.at[ring.at[m, 0]], rows[m], gsem[m])

        def _z(k, carry):
            rows[0][k // lanes, pl.ds((k % lanes) * 16, 16)] = zeros16
            return carry

        lax.fori_loop(0, _K * lanes, _z, 0)
        row0 = s * rpt
        for i in range(nz):
            pltpu.sync_copy(rows[0], acc_sh.at[pl.ds(row0 + i * _K, _K)])
        plsc.subcore_barrier()
        pltpu.async_copy(y_hbm.at[ring.at[0, 0]], rows[0], gsem[0])

        def _sub(g, j, skip_a=False, skip_b=False):
            m = j % 4
            if not skip_a:
                pltpu.make_async_copy(rows[m], acc_sh.at[ring.at[0, 1]],
                                      ssem[m]).wait()
                pltpu.make_async_copy(idx_hbm.at[w, 0], ring.at[0],
                                      isem[m]).wait()
                gn = jnp.minimum(g + 4, nchunk - 1)
                pltpu.async_copy(idx_hbm.at[w, gn], ring.at[(j + 4) % 8],
                                 isem[m])
                pltpu.async_copy(y_hbm.at[ring.at[j, 0]], rows[m], gsem[m])
            if not skip_b:
                m2 = (j + 2) % 4
                pltpu.make_async_copy(y_hbm.at[ring.at[0, 0]], rows[m2],
                                      gsem[m2]).wait()
                pltpu.async_copy(rows[m2], acc_sh.at[ring.at[(j + 6) % 8, 1]],
                                 ssem[m2], add=True)

        for j in range(8):
            _sub(j, j, skip_a=(j < 4), skip_b=(j < 2))

        def _oct(q, carry):
            for j in range(8):
                _sub(8 * q + j, j, False)
            return carry

        lax.fori_loop(1, noct, _oct, 0)
        for (m2, slot) in ((2, 6), (3, 7)):
            pltpu.make_async_copy(y_hbm.at[ring.at[0, 0]], rows[m2],
                                  gsem[m2]).wait()
            pltpu.async_copy(rows[m2], acc_sh.at[ring.at[slot, 1]],
                             ssem[m2], add=True)
        for m in range(4):
            pltpu.make_async_copy(rows[m], acc_sh.at[ring.at[0, 1]],
                                  ssem[m]).wait()
            pltpu.make_async_copy(idx_hbm.at[w, 0], ring.at[0],
                                  isem[m]).wait()
        plsc.subcore_barrier()
        pltpu.sync_copy(acc_sh.at[pl.ds(row0, rpt)],
                        out_hbm.at[c, pl.ds(row0, rpt)])

    return edge_kernel


def _matmul_t(a, w):
    return lax.dot_general(a, w, (((1,), (1,)), ((), ())),
                           preferred_element_type=jnp.float32)


def _tc1_body(x_ref, wt_ref, d0_ref, d1_ref, y_ref):
    dinv = lax.rsqrt(d0_ref[...] + d1_ref[...] + 1.0)
    y_ref[...] = dinv * _matmul_t(x_ref[...], wt_ref[...])


def _tc2_body(s0_ref, s1_ref, y_ref, wt_ref, b_ref, d0_ref, d1_ref, o_ref):
    dinv = lax.rsqrt(d0_ref[...] + d1_ref[...] + 1.0)
    o1 = dinv * (s0_ref[...] + s1_ref[...] + y_ref[...]) + b_ref[...]
    o_ref[...] = dinv * _matmul_t(o1, wt_ref[...])


def _tc3_body(s0_ref, s1_ref, y_ref, b_ref, d0_ref, d1_ref, o_ref):
    dinv = lax.rsqrt(d0_ref[...] + d1_ref[...] + 1.0)
    o_ref[...] = dinv * (s0_ref[...] + s1_ref[...] + y_ref[...]) + b_ref[...]


def _row_spec(d):
    return pl.BlockSpec((_BLK, d), lambda i: (i, 0))


def _full_spec(shape):
    return pl.BlockSpec(shape, lambda i: tuple(0 for _ in shape))


def _tc1(x, wt, d0, d1, np_rows):
    _, d = x.shape
    return pl.pallas_call(
        _tc1_body,
        grid=(np_rows // _BLK,),
        in_specs=[_row_spec(d), _full_spec((d, d)),
                  _row_spec(1), _row_spec(1)],
        out_specs=_row_spec(d),
        out_shape=jax.ShapeDtypeStruct((np_rows, d), jnp.float32),
    )(x, wt, d0, d1)


def _tc2(s0, s1, y, wt, b, d0, d1):
    np_rows, d = y.shape
    return pl.pallas_call(
        _tc2_body,
        grid=(np_rows // _BLK,),
        in_specs=[_row_spec(d), _row_spec(d), _row_spec(d),
                  _full_spec((d, d)), _full_spec((1, d)),
                  _row_spec(1), _row_spec(1)],
        out_specs=_row_spec(d),
        out_shape=jax.ShapeDtypeStruct((np_rows, d), jnp.float32),
    )(s0, s1, y, wt, b, d0, d1)


def _tc3(s0, s1, y, b, d0, d1, n):
    np_rows, d = y.shape
    return pl.pallas_call(
        _tc3_body,
        grid=(np_rows // _BLK,),
        in_specs=[_row_spec(d), _row_spec(d), _row_spec(d),
                  _full_spec((1, d)), _row_spec(1), _row_spec(1)],
        out_specs=_row_spec(d),
        out_shape=jax.ShapeDtypeStruct((n, d), jnp.float32),
    )(s0, s1, y, b, d0, d1)


def kernel(x, edge_index, W1, b1, W2, b2):
    n, d = x.shape
    e = edge_index.shape[1]
    np_rows = -(-n // _BLK) * _BLK
    if np_rows == n:
        np_rows += _BLK
    nw = 32
    ept = -(-e // nw // (8 * _K)) * (8 * _K)
    e_pad = ept * nw
    nchunk = ept // _K

    src = edge_index[0]
    dst = edge_index[1]
    if e_pad > e:
        pad = e_pad - e
        src = jnp.concatenate([src, (jnp.arange(pad, dtype=jnp.int32) % n)])
        dst = jnp.concatenate(
            [dst, n + (jnp.arange(pad, dtype=jnp.int32) % (np_rows - n))])
    src3 = src.reshape(nw, nchunk, _K)
    dst3 = dst.reshape(nw, nchunk, _K)
    idx4 = jnp.stack([src3, dst3], axis=2)

    degp = _make_deg_kernel(nchunk, np_rows)(dst3)
    d0 = degp[0].reshape(np_rows, 1)
    d1 = degp[1].reshape(np_rows, 1)

    edge_k = _make_edge_kernel(nchunk, np_rows, d)

    y1 = _tc1(x, W1, d0, d1, np_rows)
    s1 = edge_k(idx4, y1)
    y2 = _tc2(s1[0], s1[1], y1, W2, b1.reshape(1, d), d0, d1)
    s2 = edge_k(idx4, y2)
    return _tc3(s2[0], s2[1], y2, b2.reshape(1, d), d0, d1, n)


# speedup vs baseline: 33.9170x; 1.0472x over previous
import functools

import jax
import jax.numpy as jnp
from jax import lax
from jax.experimental import pallas as pl
from jax.experimental.pallas import tpu as pltpu
from jax.experimental.pallas import tpu_sc as plsc

_K = 64
_BLK = 1024


def _sc_mesh():
    return plsc.VectorSubcoreMesh(core_axis_name="c", subcore_axis_name="s")


def _make_deg_kernel(nchunk, np_rows):
    mesh = _sc_mesh()
    nc, ns = mesh.num_cores, mesh.num_subcores
    rpt = np_rows // ns
    grp = 8
    ngrp = nchunk // grp

    @functools.partial(
        pl.kernel,
        out_type=jax.ShapeDtypeStruct((nc, np_rows), jnp.float32),
        mesh=mesh,
        scratch_types=[
            pltpu.VMEM((nchunk, _K), jnp.int32),
            pltpu.VMEM((_K,), jnp.float32),
            pltpu.VMEM((rpt,), jnp.float32),
            pltpu.VMEM_SHARED((np_rows,), jnp.float32),
            pltpu.SemaphoreType.DMA,
        ],
    )
    def deg_kernel(dst_hbm, out_hbm, didx_v, ones_v, zbuf_v, deg_sh, dsem):
        c = lax.axis_index("c")
        s = lax.axis_index("s")
        w = s * nc + c
        zeros16 = jnp.zeros((16,), jnp.float32)
        ones16 = jnp.ones((16,), jnp.float32)

        def _z(k, carry):
            zbuf_v[pl.ds(k * 16, 16)] = zeros16
            return carry

        lax.fori_loop(0, rpt // 16, _z, 0)
        for j in range(_K // 16):
            ones_v[pl.ds(j * 16, 16)] = ones16
        row0 = s * rpt
        pltpu.sync_copy(zbuf_v, deg_sh.at[pl.ds(row0, rpt)])
        pltpu.sync_copy(dst_hbm.at[w], didx_v)
        plsc.subcore_barrier()

        def _grp(p, carry):
            for j in range(grp):
                pltpu.async_copy(ones_v, deg_sh.at[didx_v.at[p * grp + j]],
                                 dsem, add=True)
            for j in range(grp):
                pltpu.make_async_copy(ones_v, deg_sh.at[didx_v.at[0]],
                                      dsem).wait()
            return carry

        lax.fori_loop(0, ngrp, _grp, 0)
        plsc.subcore_barrier()
        pltpu.sync_copy(deg_sh.at[pl.ds(row0, rpt)],
                        out_hbm.at[c, pl.ds(row0, rpt)])

    return deg_kernel


def _make_edge_kernel(nchunk, np_rows, d):
    mesh = _sc_mesh()
    nc, ns = mesh.num_cores, mesh.num_subcores
    rpt = np_rows // ns
    nz = rpt // _K
    ndec = nchunk // 10
    nb = 5

    @functools.partial(
        pl.kernel,
        out_type=jax.ShapeDtypeStruct((nc, np_rows, d), jnp.float32),
        mesh=mesh,
        scratch_types=[
            pltpu.VMEM((10, 2, _K), jnp.int32),
            [pltpu.VMEM((_K, d), jnp.float32) for _ in range(nb)],
            pltpu.VMEM_SHARED((np_rows, d), jnp.float32),
            [pltpu.SemaphoreType.DMA for _ in range(nb)],
            [pltpu.SemaphoreType.DMA for _ in range(nb)],
            [pltpu.SemaphoreType.DMA for _ in range(nb)],
        ],
    )
    def edge_kernel(idx_hbm, y_hbm, out_hbm,
                    ring, rows, acc_sh, isem, gsem, ssem):
        c = lax.axis_index("c")
        s = lax.axis_index("s")
        w = s * nc + c
        zeros16 = jnp.zeros((16,), jnp.float32)
        lanes = d // 16

        for j in range(nb):
            pltpu.sync_copy(idx_hbm.at[w, j], ring.at[j])
        for j in range(nb):
            pltpu.async_copy(idx_hbm.at[w, nb + j], ring.at[nb + j],
                             isem[j])
        for m in range(1, nb):
            pltpu.async_copy(y_hbm.at[ring.at[m, 0]], rows[m], gsem[m])

        def _z(k, carry):
            rows[0][k // lanes, pl.ds((k % lanes) * 16, 16)] = zeros16
            return carry

        lax.fori_loop(0, _K * lanes, _z, 0)
        row0 = s * rpt
        for i in range(nz):
            pltpu.sync_copy(rows[0], acc_sh.at[pl.ds(row0 + i * _K, _K)])
        plsc.subcore_barrier()
        pltpu.async_copy(y_hbm.at[ring.at[0, 0]], rows[0], gsem[0])

        def _sub(g, j, skip_a=False, skip_b=False):
            m = j % nb
            if not skip_a:
                pltpu.make_async_copy(rows[m], acc_sh.at[ring.at[0, 1]],
                                      ssem[m]).wait()
                pltpu.make_async_copy(idx_hbm.at[w, 0], ring.at[0],
                                      isem[m]).wait()
                gn = jnp.minimum(g + nb, nchunk - 1)
                pltpu.async_copy(idx_hbm.at[w, gn], ring.at[(j + nb) % 10],
                                 isem[m])
                pltpu.async_copy(y_hbm.at[ring.at[j, 0]], rows[m], gsem[m])
            if not skip_b:
                m2 = (j + 2) % nb
                pltpu.make_async_copy(y_hbm.at[ring.at[0, 0]], rows[m2],
                                      gsem[m2]).wait()
                pltpu.async_copy(rows[m2], acc_sh.at[ring.at[(j + 7) % 10, 1]],
                                 ssem[m2], add=True)

        for j in range(10):
            _sub(j, j, skip_a=(j < nb), skip_b=(j < 3))

        def _dec(q, carry):
            for j in range(10):
                _sub(10 * q + j, j, False)
            return carry

        lax.fori_loop(1, ndec, _dec, 0)
        for (m2, slot) in ((2, 7), (3, 8), (4, 9)):
            pltpu.make_async_copy(y_hbm.at[ring.at[0, 0]], rows[m2],
                                  gsem[m2]).wait()
            pltpu.async_copy(rows[m2], acc_sh.at[ring.at[slot, 1]],
                             ssem[m2], add=True)
        for m in range(nb):
            pltpu.make_async_copy(rows[m], acc_sh.at[ring.at[0, 1]],
                                  ssem[m]).wait()
            pltpu.make_async_copy(idx_hbm.at[w, 0], ring.at[0],
                                  isem[m]).wait()
        plsc.subcore_barrier()
        pltpu.sync_copy(acc_sh.at[pl.ds(row0, rpt)],
                        out_hbm.at[c, pl.ds(row0, rpt)])

    return edge_kernel


def _matmul_t(a, w):
    return lax.dot_general(a, w, (((1,), (1,)), ((), ())),
                           preferred_element_type=jnp.float32)


def _tc1_body(x_ref, wt_ref, d0_ref, d1_ref, y_ref):
    dinv = lax.rsqrt(d0_ref[...] + d1_ref[...] + 1.0)
    y_ref[...] = dinv * _matmul_t(x_ref[...], wt_ref[...])


def _tc2_body(s0_ref, s1_ref, y_ref, wt_ref, b_ref, d0_ref, d1_ref, o_ref):
    dinv = lax.rsqrt(d0_ref[...] + d1_ref[...] + 1.0)
    o1 = dinv * (s0_ref[...] + s1_ref[...] + y_ref[...]) + b_ref[...]
    o_ref[...] = dinv * _matmul_t(o1, wt_ref[...])


def _tc3_body(s0_ref, s1_ref, y_ref, b_ref, d0_ref, d1_ref, o_ref):
    dinv = lax.rsqrt(d0_ref[...] + d1_ref[...] + 1.0)
    o_ref[...] = dinv * (s0_ref[...] + s1_ref[...] + y_ref[...]) + b_ref[...]


def _row_spec(d):
    return pl.BlockSpec((_BLK, d), lambda i: (i, 0))


def _full_spec(shape):
    return pl.BlockSpec(shape, lambda i: tuple(0 for _ in shape))


def _tc1(x, wt, d0, d1, np_rows):
    _, d = x.shape
    return pl.pallas_call(
        _tc1_body,
        grid=(np_rows // _BLK,),
        in_specs=[_row_spec(d), _full_spec((d, d)),
                  _row_spec(1), _row_spec(1)],
        out_specs=_row_spec(d),
        out_shape=jax.ShapeDtypeStruct((np_rows, d), jnp.float32),
    )(x, wt, d0, d1)


def _tc2(s0, s1, y, wt, b, d0, d1):
    np_rows, d = y.shape
    return pl.pallas_call(
        _tc2_body,
        grid=(np_rows // _BLK,),
        in_specs=[_row_spec(d), _row_spec(d), _row_spec(d),
                  _full_spec((d, d)), _full_spec((1, d)),
                  _row_spec(1), _row_spec(1)],
        out_specs=_row_spec(d),
        out_shape=jax.ShapeDtypeStruct((np_rows, d), jnp.float32),
    )(s0, s1, y, wt, b, d0, d1)


def _tc3(s0, s1, y, b, d0, d1, n):
    np_rows, d = y.shape
    return pl.pallas_call(
        _tc3_body,
        grid=(np_rows // _BLK,),
        in_specs=[_row_spec(d), _row_spec(d), _row_spec(d),
                  _full_spec((1, d)), _row_spec(1), _row_spec(1)],
        out_specs=_row_spec(d),
        out_shape=jax.ShapeDtypeStruct((n, d), jnp.float32),
    )(s0, s1, y, b, d0, d1)


def kernel(x, edge_index, W1, b1, W2, b2):
    n, d = x.shape
    e = edge_index.shape[1]
    np_rows = -(-n // _BLK) * _BLK
    if np_rows == n:
        np_rows += _BLK
    nw = 32
    ept = -(-e // nw // (10 * _K)) * (10 * _K)
    e_pad = ept * nw
    nchunk = ept // _K

    src = edge_index[0]
    dst = edge_index[1]
    if e_pad > e:
        pad = e_pad - e
        src = jnp.concatenate([src, (jnp.arange(pad, dtype=jnp.int32) % n)])
        dst = jnp.concatenate(
            [dst, n + (jnp.arange(pad, dtype=jnp.int32) % (np_rows - n))])
    src3 = src.reshape(nw, nchunk, _K)
    dst3 = dst.reshape(nw, nchunk, _K)
    idx4 = jnp.stack([src3, dst3], axis=2)

    degp = _make_deg_kernel(nchunk, np_rows)(dst3)
    d0 = degp[0].reshape(np_rows, 1)
    d1 = degp[1].reshape(np_rows, 1)

    edge_k = _make_edge_kernel(nchunk, np_rows, d)

    y1 = _tc1(x, W1, d0, d1, np_rows)
    s1 = edge_k(idx4, y1)
    y2 = _tc2(s1[0], s1[1], y1, W2, b1.reshape(1, d), d0, d1)
    s2 = edge_k(idx4, y2)
    return _tc3(s2[0], s2[1], y2, b2.reshape(1, d), d0, d1, n)


# speedup vs baseline: 35.0648x; 1.0338x over previous
import functools

import jax
import jax.numpy as jnp
from jax import lax
from jax.experimental import pallas as pl
from jax.experimental.pallas import tpu as pltpu
from jax.experimental.pallas import tpu_sc as plsc

_K = 64
_BLK = 1024


def _sc_mesh():
    return plsc.VectorSubcoreMesh(core_axis_name="c", subcore_axis_name="s")


def _make_deg_kernel(nchunk, np_rows):
    mesh = _sc_mesh()
    nc, ns = mesh.num_cores, mesh.num_subcores
    rpt = np_rows // ns
    grp = 8
    ngrp = nchunk // grp

    @functools.partial(
        pl.kernel,
        out_type=jax.ShapeDtypeStruct((nc, np_rows), jnp.float32),
        mesh=mesh,
        scratch_types=[
            pltpu.VMEM((nchunk, _K), jnp.int32),
            pltpu.VMEM((_K,), jnp.float32),
            pltpu.VMEM((rpt,), jnp.float32),
            pltpu.VMEM_SHARED((np_rows,), jnp.float32),
            pltpu.SemaphoreType.DMA,
        ],
    )
    def deg_kernel(dst_hbm, out_hbm, didx_v, ones_v, zbuf_v, deg_sh, dsem):
        c = lax.axis_index("c")
        s = lax.axis_index("s")
        w = s * nc + c
        zeros16 = jnp.zeros((16,), jnp.float32)
        ones16 = jnp.ones((16,), jnp.float32)

        def _z(k, carry):
            zbuf_v[pl.ds(k * 16, 16)] = zeros16
            return carry

        lax.fori_loop(0, rpt // 16, _z, 0)
        for j in range(_K // 16):
            ones_v[pl.ds(j * 16, 16)] = ones16
        row0 = s * rpt
        pltpu.sync_copy(zbuf_v, deg_sh.at[pl.ds(row0, rpt)])
        pltpu.sync_copy(dst_hbm.at[w], didx_v)
        plsc.subcore_barrier()

        def _grp(p, carry):
            for j in range(grp):
                pltpu.async_copy(ones_v, deg_sh.at[didx_v.at[p * grp + j]],
                                 dsem, add=True)
            for j in range(grp):
                pltpu.make_async_copy(ones_v, deg_sh.at[didx_v.at[0]],
                                      dsem).wait()
            return carry

        lax.fori_loop(0, ngrp, _grp, 0)
        plsc.subcore_barrier()
        pltpu.sync_copy(deg_sh.at[pl.ds(row0, rpt)],
                        out_hbm.at[c, pl.ds(row0, rpt)])

    return deg_kernel


def _make_edge_kernel(nchunk, np_rows, d):
    mesh = _sc_mesh()
    nc, ns = mesh.num_cores, mesh.num_subcores
    rpt = np_rows // ns
    nz = rpt // _K
    ndec = nchunk // 10
    nb = 5

    @functools.partial(
        pl.kernel,
        out_type=jax.ShapeDtypeStruct((nc, np_rows, d), jnp.float32),
        mesh=mesh,
        scratch_types=[
            pltpu.VMEM((10, 2, _K), jnp.int32),
            [pltpu.VMEM((_K, d), jnp.float32) for _ in range(nb)],
            pltpu.VMEM_SHARED((np_rows, d), jnp.float32),
            [pltpu.SemaphoreType.DMA for _ in range(nb)],
            [pltpu.SemaphoreType.DMA for _ in range(nb)],
            [pltpu.SemaphoreType.DMA for _ in range(nb)],
        ],
    )
    def edge_kernel(idx_hbm, y_hbm, out_hbm,
                    ring, rows, acc_sh, isem, gsem, ssem):
        c = lax.axis_index("c")
        s = lax.axis_index("s")
        w = s * nc + c
        zeros16 = jnp.zeros((16,), jnp.float32)
        lanes = d // 16

        for j in range(nb):
            pltpu.sync_copy(idx_hbm.at[w, j], ring.at[j])
        for j in range(nb):
            pltpu.async_copy(idx_hbm.at[w, nb + j], ring.at[nb + j],
                             isem[j])
        for m in range(1, nb):
            pltpu.async_copy(y_hbm.at[ring.at[m, 0]], rows[m], gsem[m])

        def _z(k, carry):
            rows[0][k // lanes, pl.ds((k % lanes) * 16, 16)] = zeros16
            return carry

        lax.fori_loop(0, _K * lanes, _z, 0)
        row0 = s * rpt
        for i in range(nz):
            pltpu.sync_copy(rows[0], acc_sh.at[pl.ds(row0 + i * _K, _K)])
        plsc.subcore_barrier()
        pltpu.async_copy(y_hbm.at[ring.at[0, 0]], rows[0], gsem[0])

        def _sub(g, j, skip_a=False, skip_b=False):
            m = j % nb
            if not skip_a:
                pltpu.make_async_copy(rows[m], acc_sh.at[ring.at[0, 1]],
                                      ssem[m]).wait()
                pltpu.make_async_copy(idx_hbm.at[w, 0], ring.at[0],
                                      isem[m]).wait()
                gn = jnp.minimum(g + nb, nchunk - 1)
                pltpu.async_copy(idx_hbm.at[w, gn], ring.at[(j + nb) % 10],
                                 isem[m])
                pltpu.async_copy(y_hbm.at[ring.at[j, 0]], rows[m], gsem[m])
            if not skip_b:
                m2 = (j + 1) % nb
                pltpu.make_async_copy(y_hbm.at[ring.at[0, 0]], rows[m2],
                                      gsem[m2]).wait()
                pltpu.async_copy(rows[m2], acc_sh.at[ring.at[(j + 6) % 10, 1]],
                                 ssem[m2], add=True)

        for j in range(10):
            _sub(j, j, skip_a=(j < nb), skip_b=(j < 4))

        def _dec(q, carry):
            for j in range(10):
                _sub(10 * q + j, j, False)
            return carry

        lax.fori_loop(1, ndec, _dec, 0)
        for (m2, slot) in ((1, 6), (2, 7), (3, 8), (4, 9)):
            pltpu.make_async_copy(y_hbm.at[ring.at[0, 0]], rows[m2],
                                  gsem[m2]).wait()
            pltpu.async_copy(rows[m2], acc_sh.at[ring.at[slot, 1]],
                             ssem[m2], add=True)
        for m in range(nb):
            pltpu.make_async_copy(rows[m], acc_sh.at[ring.at[0, 1]],
                                  ssem[m]).wait()
            pltpu.make_async_copy(idx_hbm.at[w, 0], ring.at[0],
                                  isem[m]).wait()
        plsc.subcore_barrier()
        pltpu.sync_copy(acc_sh.at[pl.ds(row0, rpt)],
                        out_hbm.at[c, pl.ds(row0, rpt)])

    return edge_kernel


def _matmul_t(a, w):
    return lax.dot_general(a, w, (((1,), (1,)), ((), ())),
                           preferred_element_type=jnp.float32)


def _tc1_body(x_ref, wt_ref, d0_ref, d1_ref, y_ref):
    dinv = lax.rsqrt(d0_ref[...] + d1_ref[...] + 1.0)
    y_ref[...] = dinv * _matmul_t(x_ref[...], wt_ref[...])


def _tc2_body(s0_ref, s1_ref, y_ref, wt_ref, b_ref, d0_ref, d1_ref, o_ref):
    dinv = lax.rsqrt(d0_ref[...] + d1_ref[...] + 1.0)
    o1 = dinv * (s0_ref[...] + s1_ref[...] + y_ref[...]) + b_ref[...]
    o_ref[...] = dinv * _matmul_t(o1, wt_ref[...])


def _tc3_body(s0_ref, s1_ref, y_ref, b_ref, d0_ref, d1_ref, o_ref):
    dinv = lax.rsqrt(d0_ref[...] + d1_ref[...] + 1.0)
    o_ref[...] = dinv * (s0_ref[...] + s1_ref[...] + y_ref[...]) + b_ref[...]


def _row_spec(d):
    return pl.BlockSpec((_BLK, d), lambda i: (i, 0))


def _full_spec(shape):
    return pl.BlockSpec(shape, lambda i: tuple(0 for _ in shape))


def _tc1(x, wt, d0, d1, np_rows):
    _, d = x.shape
    return pl.pallas_call(
        _tc1_body,
        grid=(np_rows // _BLK,),
        in_specs=[_row_spec(d), _full_spec((d, d)),
                  _row_spec(1), _row_spec(1)],
        out_specs=_row_spec(d),
        out_shape=jax.ShapeDtypeStruct((np_rows, d), jnp.float32),
    )(x, wt, d0, d1)


def _tc2(s0, s1, y, wt, b, d0, d1):
    np_rows, d = y.shape
    return pl.pallas_call(
        _tc2_body,
        grid=(np_rows // _BLK,),
        in_specs=[_row_spec(d), _row_spec(d), _row_spec(d),
                  _full_spec((d, d)), _full_spec((1, d)),
                  _row_spec(1), _row_spec(1)],
        out_specs=_row_spec(d),
        out_shape=jax.ShapeDtypeStruct((np_rows, d), jnp.float32),
    )(s0, s1, y, wt, b, d0, d1)


def _tc3(s0, s1, y, b, d0, d1, n):
    np_rows, d = y.shape
    return pl.pallas_call(
        _tc3_body,
        grid=(np_rows // _BLK,),
        in_specs=[_row_spec(d), _row_spec(d), _row_spec(d),
                  _full_spec((1, d)), _row_spec(1), _row_spec(1)],
        out_specs=_row_spec(d),
        out_shape=jax.ShapeDtypeStruct((n, d), jnp.float32),
    )(s0, s1, y, b, d0, d1)


def kernel(x, edge_index, W1, b1, W2, b2):
    n, d = x.shape
    e = edge_index.shape[1]
    np_rows = -(-n // _BLK) * _BLK
    if np_rows == n:
        np_rows += _BLK
    nw = 32
    ept = -(-e // nw // (10 * _K)) * (10 * _K)
    e_pad = ept * nw
    nchunk = ept // _K

    src = edge_index[0]
    dst = edge_index[1]
    if e_pad > e:
        pad = e_pad - e
        src = jnp.concatenate([src, (jnp.arange(pad, dtype=jnp.int32) % n)])
        dst = jnp.concatenate(
            [dst, n + (jnp.arange(pad, dtype=jnp.int32) % (np_rows - n))])
    src3 = src.reshape(nw, nchunk, _K)
    dst3 = dst.reshape(nw, nchunk, _K)
    idx4 = jnp.stack([src3, dst3], axis=2)

    degp = _make_deg_kernel(nchunk, np_rows)(dst3)
    d0 = degp[0].reshape(np_rows, 1)
    d1 = degp[1].reshape(np_rows, 1)

    edge_k = _make_edge_kernel(nchunk, np_rows, d)

    y1 = _tc1(x, W1, d0, d1, np_rows)
    s1 = edge_k(idx4, y1)
    y2 = _tc2(s1[0], s1[1], y1, W2, b1.reshape(1, d), d0, d1)
    s2 = edge_k(idx4, y2)
    return _tc3(s2[0], s2[1], y2, b2.reshape(1, d), d0, d1, n)


# speedup vs baseline: 35.1056x; 1.0012x over previous
import functools

import jax
import jax.numpy as jnp
from jax import lax
from jax.experimental import pallas as pl
from jax.experimental.pallas import tpu as pltpu
from jax.experimental.pallas import tpu_sc as plsc

_K = 64
_BLK = 1024


def _sc_mesh():
    return plsc.VectorSubcoreMesh(core_axis_name="c", subcore_axis_name="s")


def _make_deg_kernel(nchunk, np_rows):
    mesh = _sc_mesh()
    nc, ns = mesh.num_cores, mesh.num_subcores
    rpt = np_rows // ns
    grp = 8
    ngrp = nchunk // grp

    @functools.partial(
        pl.kernel,
        out_type=jax.ShapeDtypeStruct((nc, np_rows), jnp.float32),
        mesh=mesh,
        scratch_types=[
            pltpu.VMEM((nchunk, _K), jnp.int32),
            pltpu.VMEM((_K,), jnp.float32),
            pltpu.VMEM((rpt,), jnp.float32),
            pltpu.VMEM_SHARED((np_rows,), jnp.float32),
            pltpu.SemaphoreType.DMA,
        ],
    )
    def deg_kernel(dst_hbm, out_hbm, didx_v, ones_v, zbuf_v, deg_sh, dsem):
        c = lax.axis_index("c")
        s = lax.axis_index("s")
        w = s * nc + c
        zeros16 = jnp.zeros((16,), jnp.float32)
        ones16 = jnp.ones((16,), jnp.float32)

        def _z(k, carry):
            zbuf_v[pl.ds(k * 16, 16)] = zeros16
            return carry

        lax.fori_loop(0, rpt // 16, _z, 0)
        for j in range(_K // 16):
            ones_v[pl.ds(j * 16, 16)] = ones16
        row0 = s * rpt
        pltpu.sync_copy(zbuf_v, deg_sh.at[pl.ds(row0, rpt)])
        pltpu.sync_copy(dst_hbm.at[w], didx_v)
        plsc.subcore_barrier()

        def _grp(p, carry):
            for j in range(grp):
                pltpu.async_copy(ones_v, deg_sh.at[didx_v.at[p * grp + j]],
                                 dsem, add=True)
            for j in range(grp):
                pltpu.make_async_copy(ones_v, deg_sh.at[didx_v.at[0]],
                                      dsem).wait()
            return carry

        lax.fori_loop(0, ngrp, _grp, 0)
        for j in range(nchunk % grp):
            pltpu.async_copy(ones_v, deg_sh.at[didx_v.at[ngrp * grp + j]],
                             dsem, add=True)
        for j in range(nchunk % grp):
            pltpu.make_async_copy(ones_v, deg_sh.at[didx_v.at[0]],
                                  dsem).wait()
        plsc.subcore_barrier()
        pltpu.sync_copy(deg_sh.at[pl.ds(row0, rpt)],
                        out_hbm.at[c, pl.ds(row0, rpt)])

    return deg_kernel


def _make_edge_kernel(nchunk, np_rows, d):
    mesh = _sc_mesh()
    nc, ns = mesh.num_cores, mesh.num_subcores
    rpt = np_rows // ns
    nz = rpt // _K
    ndec = nchunk // 10
    nb = 5

    @functools.partial(
        pl.kernel,
        out_type=jax.ShapeDtypeStruct((nc, np_rows, d), jnp.float32),
        mesh=mesh,
        scratch_types=[
            pltpu.VMEM((10, 2, _K), jnp.int32),
            [pltpu.VMEM((_K, d), jnp.float32) for _ in range(nb)],
            pltpu.VMEM_SHARED((np_rows, d), jnp.float32),
            [pltpu.SemaphoreType.DMA for _ in range(nb)],
            [pltpu.SemaphoreType.DMA for _ in range(nb)],
            [pltpu.SemaphoreType.DMA for _ in range(nb)],
        ],
    )
    def edge_kernel(idx_hbm, y_hbm, out_hbm,
                    ring, rows, acc_sh, isem, gsem, ssem):
        c = lax.axis_index("c")
        s = lax.axis_index("s")
        w = s * nc + c
        zeros16 = jnp.zeros((16,), jnp.float32)
        lanes = d // 16

        for j in range(nb):
            pltpu.sync_copy(idx_hbm.at[w, j], ring.at[j])
        for j in range(nb):
            pltpu.async_copy(idx_hbm.at[w, nb + j], ring.at[nb + j],
                             isem[j])
        for m in range(1, nb):
            pltpu.async_copy(y_hbm.at[ring.at[m, 0]], rows[m], gsem[m])

        def _z(k, carry):
            rows[0][k // lanes, pl.ds((k % lanes) * 16, 16)] = zeros16
            return carry

        lax.fori_loop(0, _K * lanes, _z, 0)
        row0 = s * rpt
        for i in range(nz):
            pltpu.sync_copy(rows[0], acc_sh.at[pl.ds(row0 + i * _K, _K)])
        plsc.subcore_barrier()
        pltpu.async_copy(y_hbm.at[ring.at[0, 0]], rows[0], gsem[0])

        def _sub(g, j, skip_a=False, skip_b=False):
            m = j % nb
            if not skip_a:
                pltpu.make_async_copy(rows[m], acc_sh.at[ring.at[0, 1]],
                                      ssem[m]).wait()
                pltpu.make_async_copy(idx_hbm.at[w, 0], ring.at[0],
                                      isem[m]).wait()
                gn = jnp.minimum(g + nb, nchunk - 1)
                pltpu.async_copy(idx_hbm.at[w, gn], ring.at[(j + nb) % 10],
                                 isem[m])
                pltpu.async_copy(y_hbm.at[ring.at[j, 0]], rows[m], gsem[m])
            if not skip_b:
                m2 = (j + 1) % nb
                pltpu.make_async_copy(y_hbm.at[ring.at[0, 0]], rows[m2],
                                      gsem[m2]).wait()
                pltpu.async_copy(rows[m2], acc_sh.at[ring.at[(j + 6) % 10, 1]],
                                 ssem[m2], add=True)

        for j in range(10):
            _sub(j, j, skip_a=(j < nb), skip_b=(j < 4))

        def _dec(q, carry):
            for j in range(10):
                _sub(10 * q + j, j, False)
            return carry

        lax.fori_loop(1, ndec, _dec, 0)
        for (m2, slot) in ((1, 6), (2, 7), (3, 8), (4, 9)):
            pltpu.make_async_copy(y_hbm.at[ring.at[0, 0]], rows[m2],
                                  gsem[m2]).wait()
            pltpu.async_copy(rows[m2], acc_sh.at[ring.at[slot, 1]],
                             ssem[m2], add=True)
        for m in range(nb):
            pltpu.make_async_copy(rows[m], acc_sh.at[ring.at[0, 1]],
                                  ssem[m]).wait()
            pltpu.make_async_copy(idx_hbm.at[w, 0], ring.at[0],
                                  isem[m]).wait()
        plsc.subcore_barrier()
        pltpu.sync_copy(acc_sh.at[pl.ds(row0, rpt)],
                        out_hbm.at[c, pl.ds(row0, rpt)])

    return edge_kernel


def _matmul_t(a, w):
    return lax.dot_general(a, w, (((1,), (1,)), ((), ())),
                           preferred_element_type=jnp.float32)


def _tc1_body(x_ref, wt_ref, d0_ref, d1_ref, y_ref):
    dinv = lax.rsqrt(d0_ref[...] + d1_ref[...] + 1.0)
    y_ref[...] = dinv * _matmul_t(x_ref[...], wt_ref[...])


def _tc2_body(s0_ref, s1_ref, y_ref, wt_ref, b_ref, d0_ref, d1_ref, o_ref):
    dinv = lax.rsqrt(d0_ref[...] + d1_ref[...] + 1.0)
    o1 = dinv * (s0_ref[...] + s1_ref[...] + y_ref[...]) + b_ref[...]
    o_ref[...] = dinv * _matmul_t(o1, wt_ref[...])


def _tc3_body(s0_ref, s1_ref, y_ref, b_ref, d0_ref, d1_ref, o_ref):
    dinv = lax.rsqrt(d0_ref[...] + d1_ref[...] + 1.0)
    o_ref[...] = dinv * (s0_ref[...] + s1_ref[...] + y_ref[...]) + b_ref[...]


def _row_spec(d):
    return pl.BlockSpec((_BLK, d), lambda i: (i, 0))


def _full_spec(shape):
    return pl.BlockSpec(shape, lambda i: tuple(0 for _ in shape))


def _tc1(x, wt, d0, d1, np_rows):
    _, d = x.shape
    return pl.pallas_call(
        _tc1_body,
        grid=(np_rows // _BLK,),
        in_specs=[_row_spec(d), _full_spec((d, d)),
                  _row_spec(1), _row_spec(1)],
        out_specs=_row_spec(d),
        out_shape=jax.ShapeDtypeStruct((np_rows, d), jnp.float32),
    )(x, wt, d0, d1)


def _tc2(s0, s1, y, wt, b, d0, d1):
    np_rows, d = y.shape
    return pl.pallas_call(
        _tc2_body,
        grid=(np_rows // _BLK,),
        in_specs=[_row_spec(d), _row_spec(d), _row_spec(d),
                  _full_spec((d, d)), _full_spec((1, d)),
                  _row_spec(1), _row_spec(1)],
        out_specs=_row_spec(d),
        out_shape=jax.ShapeDtypeStruct((np_rows, d), jnp.float32),
    )(s0, s1, y, wt, b, d0, d1)


def _tc3(s0, s1, y, b, d0, d1, n):
    np_rows, d = y.shape
    return pl.pallas_call(
        _tc3_body,
        grid=(np_rows // _BLK,),
        in_specs=[_row_spec(d), _row_spec(d), _row_spec(d),
                  _full_spec((1, d)), _row_spec(1), _row_spec(1)],
        out_specs=_row_spec(d),
        out_shape=jax.ShapeDtypeStruct((n, d), jnp.float32),
    )(s0, s1, y, b, d0, d1)


def kernel(x, edge_index, W1, b1, W2, b2):
    n, d = x.shape
    e = edge_index.shape[1]
    np_rows = -(-n // _BLK) * _BLK
    if np_rows == n:
        np_rows += _BLK
    m = _sc_mesh()
    nw = m.num_cores * m.num_subcores
    ept = -(-e // nw // (10 * _K)) * (10 * _K)
    e_pad = ept * nw
    nchunk = ept // _K

    src = edge_index[0]
    dst = edge_index[1]
    if e_pad > e:
        pad = e_pad - e
        src = jnp.concatenate([src, (jnp.arange(pad, dtype=jnp.int32) % n)])
        dst = jnp.concatenate(
            [dst, n + (jnp.arange(pad, dtype=jnp.int32) % (np_rows - n))])
    src3 = src.reshape(nw, nchunk, _K)
    dst3 = dst.reshape(nw, nchunk, _K)
    idx4 = jnp.stack([src3, dst3], axis=2)

    degp = _make_deg_kernel(nchunk, np_rows)(dst3)
    d0 = degp[0].reshape(np_rows, 1)
    d1 = degp[1].reshape(np_rows, 1)

    edge_k = _make_edge_kernel(nchunk, np_rows, d)

    y1 = _tc1(x, W1, d0, d1, np_rows)
    s1 = edge_k(idx4, y1)
    y2 = _tc2(s1[0], s1[1], y1, W2, b1.reshape(1, d), d0, d1)
    s2 = edge_k(idx4, y2)
    return _tc3(s2[0], s2[1], y2, b2.reshape(1, d), d0, d1, n)
